# Initial kernel scaffold; baseline (speedup 1.0000x reference)
import jax
import jax.numpy as jnp
from jax.experimental import pallas as pl


def kernel(x, edge_index, edge_attr, batch, gW_rel_0, gb_0, gW_root_0, gW_rel_1, gb_1, gW_root_1, gW_rel_2, gb_2, gW_root_2, gW_rel_3, gb_3, gW_root_3, lstm_Wih, lstm_Whh, lstm_bih, lstm_bhh, mlp_W0, mlp_b0, mlp_W1, mlp_b1, out_W, out_b):
    raise NotImplementedError("write your pallas kernel here")


# speedup vs baseline: 6.5555x; 6.5555x over previous
import functools

import jax
import jax.numpy as jnp
from jax import lax
from jax.experimental import pallas as pl
from jax.experimental.pallas import tpu as pltpu
from jax.experimental.pallas import tpu_sc as plsc

N = 50000
NPAD = 51200
E = 800000
EPAD = 819200
NG = 50
REPS = 10
NC, NS, L = 2, 16, 16
ROWS_PT = NPAD // NS
EROWS = EPAD // 128

_GD = lax.GatherDimensionNumbers(
    offset_dims=(), collapsed_slice_dims=(0,), start_index_map=(0,))


def _lane_bcast(v, e):
    idx = jnp.full((L, 1), e, jnp.int32)
    return lax.gather(v, idx, _GD, (1,),
                      mode=lax.GatherScatterMode.PROMISE_IN_BOUNDS)


def _make_agg(width, split_edges):
    if split_edges:
        rows_per_tile = EROWS // (NC * NS)
        ch = 8
    else:
        rows_per_tile = EROWS // NS
        ch = 4
    n_chunks = rows_per_tile // ch
    halves = width // L

    mesh = plsc.VectorSubcoreMesh(core_axis_name="c", subcore_axis_name="s")

    @functools.partial(
        pl.kernel,
        out_type=jax.ShapeDtypeStruct((NC * NPAD, width), jnp.float32),
        mesh=mesh,
        compiler_params=pltpu.CompilerParams(use_tc_tiling_on_sc=False),
        scratch_types=[
            pltpu.VMEM_SHARED((NPAD, width), jnp.float32),
            pltpu.VMEM((ch, 128), jnp.int32),
            pltpu.VMEM((ch, 128), jnp.int32),
            pltpu.VMEM((ch, 128), jnp.float32),
            pltpu.VMEM((ch, 128, width), jnp.float32),
            pltpu.SemaphoreType.DMA,
        ],
    )
    def agg_kernel(h_hbm, src_hbm, dst_hbm, w_hbm, z_hbm, out_hbm,
                   acc, idxs, idxd, wv, rows, sem):
        cid = lax.axis_index("c")
        sid = lax.axis_index("s")

        pltpu.sync_copy(z_hbm, acc.at[pl.ds(sid * ROWS_PT, ROWS_PT)])
        plsc.subcore_barrier()

        if split_edges:
            wid = cid * NS + sid
            base_row = wid * rows_per_tile
            src_row = base_row
        else:
            base_row = sid * rows_per_tile
            src_row = cid * EROWS + base_row

        def chunk(k, carry):
            r0 = base_row + k * ch
            rs = src_row + k * ch
            pltpu.sync_copy(src_hbm.at[pl.ds(rs, ch)], idxs)
            pltpu.sync_copy(dst_hbm.at[pl.ds(r0, ch)], idxd)
            pltpu.sync_copy(w_hbm.at[pl.ds(r0, ch)], wv)
            descs = [
                pltpu.async_copy(h_hbm.at[idxs.at[j]], rows.at[j], sem)
                for j in range(ch)
            ]
            for d in descs:
                d.wait()

            def scale_row(j, c0):
                def scale_grp(g, c1):
                    w16 = wv[j, pl.ds(g * L, L)]
                    for e in range(L):
                        wb = _lane_bcast(w16, e)
                        r = g * L + e
                        for hh in range(halves):
                            sl = pl.ds(hh * L, L)
                            rows[j, r, sl] = rows[j, r, sl] * wb
                    return c1
                return lax.fori_loop(0, 128 // L, scale_grp, c0)
            lax.fori_loop(0, ch, scale_row, 0)

            for j in range(ch):
                pltpu.sync_copy(rows.at[j], acc.at[idxd.at[j]], add=True)
            return carry

        lax.fori_loop(0, n_chunks, chunk, 0)
        plsc.subcore_barrier()

        r0 = sid * ROWS_PT
        pltpu.sync_copy(acc.at[pl.ds(r0, ROWS_PT)],
                        out_hbm.at[pl.ds(cid * NPAD + r0, ROWS_PT)])

    return agg_kernel


BN = 2048
NB = NPAD // BN


def _l0_body(agg_ref, x_ref, wrl, wrh, wsl, wsh, bl, bh, out_ref):
    a = agg_ref[0] + agg_ref[1]
    x = x_ref[...]
    out_ref[0] = jnp.maximum(
        jnp.dot(a, wrl[...], preferred_element_type=jnp.float32)
        + jnp.dot(x, wsl[...], preferred_element_type=jnp.float32) + bl[...], 0.0)
    out_ref[1] = jnp.maximum(
        jnp.dot(a, wrh[...], preferred_element_type=jnp.float32)
        + jnp.dot(x, wsh[...], preferred_element_type=jnp.float32) + bh[...], 0.0)


def _conv_body(agg_ref, h_ref, wr_ll, wr_lh, wr_hl, wr_hh,
               ws_ll, ws_lh, ws_hl, ws_hh, bl, bh, out_ref):
    a0, a1 = agg_ref[0], agg_ref[1]
    h0, h1 = h_ref[0], h_ref[1]
    dot = lambda m, w: jnp.dot(m, w[...], preferred_element_type=jnp.float32)
    out_ref[0] = jnp.maximum(
        dot(a0, wr_ll) + dot(a1, wr_hl) + dot(h0, ws_ll) + dot(h1, ws_hl)
        + bl[...], 0.0)
    out_ref[1] = jnp.maximum(
        dot(a0, wr_lh) + dot(a1, wr_hh) + dot(h0, ws_lh) + dot(h1, ws_hh)
        + bh[...], 0.0)


def _sigmoid(x):
    return 1.0 / (1.0 + jnp.exp(-x))


def _head_body(h_ref, batch_ref,
               wi, wf, wg, wo, ui, uf, ug, uo, bi, bf, bg, bo,
               w0, b0, w1, b1, wout, bout,
               out_ref, sums, cnt):
    i = pl.program_id(0)

    @pl.when(i == 0)
    def _init():
        sums[...] = jnp.zeros_like(sums)
        cnt[...] = jnp.zeros_like(cnt)

    hcat = jnp.concatenate([h_ref[0], h_ref[1]], axis=1)
    brow = batch_ref[0]
    oh = (lax.broadcasted_iota(jnp.int32, (128, BN), 0) == brow
          ).astype(jnp.float32)
    sums[...] += jnp.dot(oh, hcat, preferred_element_type=jnp.float32)
    cnt[...] += jnp.dot(oh, jnp.ones((BN, 8), jnp.float32),
                        preferred_element_type=jnp.float32)

    @pl.when(i == NB - 1)
    def _head():
        pooled = sums[...] / jnp.maximum(cnt[:, 0:1], 1.0)
        riota = lax.broadcasted_iota(jnp.int32, (8, 128), 0)
        ciota = lax.broadcasted_iota(jnp.int32, (8, 128), 1)
        dot = lambda m, w: jnp.dot(m, w[...], preferred_element_type=jnp.float32)
        hh = jnp.zeros((8, 64), jnp.float32)
        cc = jnp.zeros((8, 64), jnp.float32)
        for t in range(REPS):
            sel = (riota * REPS + t == ciota).astype(jnp.float32)
            xt = jnp.dot(sel, pooled, preferred_element_type=jnp.float32)
            gi = _sigmoid(dot(xt, wi) + dot(hh, ui) + bi[...])
            gf = _sigmoid(dot(xt, wf) + dot(hh, uf) + bf[...])
            gg = jnp.tanh(dot(xt, wg) + dot(hh, ug) + bg[...])
            go = _sigmoid(dot(xt, wo) + dot(hh, uo) + bo[...])
            cc = gf * cc + gi * gg
            hh = go * jnp.tanh(cc)
            y = dot(hh, w0) + b0[...]
            y = dot(y, w1) + b1[...]
            y = dot(y, wout) + bout[...]
            out_ref[pl.ds(8 * t, 8), :] = y


def _full_spec(shape):
    return pl.BlockSpec(shape, lambda i: tuple(0 for _ in shape))


def _tc_layer0(agg0, x16, wrl, wrh, wsl, wsh, bl, bh):
    return pl.pallas_call(
        _l0_body,
        grid=(NB,),
        in_specs=[
            pl.BlockSpec((2, BN, 16), lambda i: (0, i, 0)),
            pl.BlockSpec((BN, 16), lambda i: (i, 0)),
            _full_spec((16, 32)), _full_spec((16, 32)),
            _full_spec((16, 32)), _full_spec((16, 32)),
            _full_spec((1, 32)), _full_spec((1, 32)),
        ],
        out_specs=pl.BlockSpec((2, BN, 32), lambda i: (0, i, 0)),
        out_shape=jax.ShapeDtypeStruct((2, NPAD, 32), jnp.float32),
    )(agg0, x16, wrl, wrh, wsl, wsh, bl, bh)


def _tc_conv(agg, h, wq, sq, bl, bh):
    return pl.pallas_call(
        _conv_body,
        grid=(NB,),
        in_specs=[
            pl.BlockSpec((2, BN, 32), lambda i: (0, i, 0)),
            pl.BlockSpec((2, BN, 32), lambda i: (0, i, 0)),
        ] + [_full_spec((32, 32))] * 8 + [_full_spec((1, 32))] * 2,
        out_specs=pl.BlockSpec((2, BN, 32), lambda i: (0, i, 0)),
        out_shape=jax.ShapeDtypeStruct((2, NPAD, 32), jnp.float32),
    )(agg, h, *wq, *sq, bl, bh)


def _tc_head(h, batch2d, lstm_w, mlp_w):
    return pl.pallas_call(
        _head_body,
        grid=(NB,),
        in_specs=[
            pl.BlockSpec((2, BN, 32), lambda i: (0, i, 0)),
            pl.BlockSpec((1, 1, BN), lambda i: (i, 0, 0)),
        ] + [_full_spec((64, 64))] * 8 + [_full_spec((1, 64))] * 4
          + [_full_spec((64, 64)), _full_spec((1, 64)),
             _full_spec((64, 32)), _full_spec((1, 32)),
             _full_spec((32, 128)), _full_spec((1, 128))],
        out_specs=pl.BlockSpec((80, 128), lambda i: (0, 0)),
        out_shape=jax.ShapeDtypeStruct((80, 128), jnp.float32),
        scratch_shapes=[
            pltpu.VMEM((128, 64), jnp.float32),
            pltpu.VMEM((128, 8), jnp.float32),
        ],
    )(h, batch2d, *lstm_w, *mlp_w)


def kernel(x, edge_index, edge_attr, batch,
           gW_rel_0, gb_0, gW_root_0,
           gW_rel_1, gb_1, gW_root_1,
           gW_rel_2, gb_2, gW_root_2,
           gW_rel_3, gb_3, gW_root_3,
           lstm_Wih, lstm_Whh, lstm_bih, lstm_bhh,
           mlp_W0, mlp_b0, mlp_W1, mlp_b1, out_W, out_b):
    f32 = jnp.float32
    src = edge_index[0]
    dst = edge_index[1]
    npad = EPAD - E
    pad_idx = (jnp.arange(npad, dtype=jnp.int32) * 41) % N
    srcp = jnp.concatenate([src, pad_idx])
    dstp = jnp.concatenate([dst, pad_idx])
    wp = jnp.concatenate([edge_attr, jnp.zeros((npad,), f32)])
    src2 = jnp.concatenate([srcp, srcp + NPAD]).reshape(2 * EROWS, 128)
    dst2 = dstp.reshape(EROWS, 128)
    w2 = wp.reshape(EROWS, 128)

    x16 = jnp.pad(x, ((0, NPAD - N), (0, 16 - x.shape[1])))
    z16 = jnp.zeros((ROWS_PT, 16), f32)
    z32 = jnp.zeros((ROWS_PT, 32), f32)

    agg16_fn = _make_agg(16, split_edges=True)
    agg32_fn = _make_agg(32, split_edges=False)

    agg0 = agg16_fn(x16, src2, dst2, w2, z16).reshape(2, NPAD, 16)
    wr0 = jnp.pad(gW_rel_0, ((0, 13), (0, 0)))
    ws0 = jnp.pad(gW_root_0, ((0, 13), (0, 0)))
    h = _tc_layer0(agg0, x16,
                   wr0[:, :32], wr0[:, 32:], ws0[:, :32], ws0[:, 32:],
                   gb_0[:32].reshape(1, 32), gb_0[32:].reshape(1, 32))

    for wr, ws, b in ((gW_rel_1, gW_root_1, gb_1),
                      (gW_rel_2, gW_root_2, gb_2),
                      (gW_rel_3, gW_root_3, gb_3)):
        agg = agg32_fn(h.reshape(2 * NPAD, 32), src2, dst2, w2, z32)
        wq = (wr[:32, :32], wr[:32, 32:], wr[32:, :32], wr[32:, 32:])
        sq = (ws[:32, :32], ws[:32, 32:], ws[32:, :32], ws[32:, 32:])
        h = _tc_conv(agg.reshape(2, NPAD, 32), h, wq, sq,
                     b[:32].reshape(1, 32), b[32:].reshape(1, 32))

    wih_t = lstm_Wih.T
    whh_t = lstm_Whh.T
    bsum = (lstm_bih + lstm_bhh).reshape(1, 256)
    lstm_w = (wih_t[:, 0:64], wih_t[:, 64:128], wih_t[:, 128:192],
              wih_t[:, 192:256],
              whh_t[:, 0:64], whh_t[:, 64:128], whh_t[:, 128:192],
              whh_t[:, 192:256],
              bsum[:, 0:64], bsum[:, 64:128], bsum[:, 128:192],
              bsum[:, 192:256])
    wout = jnp.zeros((32, 128), f32).at[:, 0:1].set(out_W)
    bout = jnp.zeros((1, 128), f32).at[0, 0].set(out_b[0])
    mlp_w = (mlp_W0, mlp_b0.reshape(1, 64), mlp_W1, mlp_b1.reshape(1, 32),
             wout, bout)
    batch_pad = jnp.concatenate(
        [batch, jnp.full((NPAD - N,), 100, jnp.int32)])
    out80 = _tc_head(h, batch_pad.reshape(NB, 1, BN), lstm_w, mlp_w)

    y = out80.reshape(REPS, 8, 128)[:, :NG // REPS, 0]
    return jnp.transpose(y)[:, :, None]


# speedup vs baseline: 9.2004x; 1.4035x over previous
import functools

import jax
import jax.numpy as jnp
from jax import lax
from jax.experimental import pallas as pl
from jax.experimental.pallas import tpu as pltpu
from jax.experimental.pallas import tpu_sc as plsc

N = 50000
NPAD = 51200
E = 800000
EPAD = 819200
NG = 50
REPS = 10
NC, NS, L = 2, 16, 16
ROWS_PT = NPAD // NS
EROWS = EPAD // 128

_GD = lax.GatherDimensionNumbers(
    offset_dims=(), collapsed_slice_dims=(0,), start_index_map=(0,))


def _lane_bcast(v, e):
    idx = jnp.full((L, 1), e, jnp.int32)
    return lax.gather(v, idx, _GD, (1,),
                      mode=lax.GatherScatterMode.PROMISE_IN_BOUNDS)


def _make_agg(width, split_edges):
    if split_edges:
        rows_per_tile = EROWS // (NC * NS)
        ch = 8
    else:
        rows_per_tile = EROWS // NS
        ch = 4
    n_chunks = rows_per_tile // ch
    halves = width // L

    mesh = plsc.VectorSubcoreMesh(core_axis_name="c", subcore_axis_name="s")

    @functools.partial(
        pl.kernel,
        out_type=jax.ShapeDtypeStruct((NC * NPAD, width), jnp.float32),
        mesh=mesh,
        compiler_params=pltpu.CompilerParams(use_tc_tiling_on_sc=False),
        scratch_types=[
            pltpu.VMEM_SHARED((NPAD, width), jnp.float32),
            pltpu.VMEM((ch, 128), jnp.int32),
            pltpu.VMEM((ch, 128), jnp.int32),
            pltpu.VMEM((ch, 128), jnp.float32),
            pltpu.VMEM((ch * 128, width), jnp.float32),
            pltpu.SemaphoreType.DMA,
            pltpu.SemaphoreType.DMA,
            pltpu.SemaphoreType.DMA,
            pltpu.SemaphoreType.DMA,
        ],
    )
    def agg_kernel(h_hbm, src_hbm, dst_hbm, w_hbm, z_hbm, out_hbm,
                   acc, idxs, idxd, wv, rows, sem_i, sem_g, sem_sa, sem_sb):
        cid = lax.axis_index("c")
        sid = lax.axis_index("s")

        pltpu.sync_copy(z_hbm, acc.at[pl.ds(sid * ROWS_PT, ROWS_PT)])
        plsc.subcore_barrier()

        if split_edges:
            wid = cid * NS + sid
            base_row = wid * rows_per_tile
            src_row = base_row
        else:
            base_row = sid * rows_per_tile
            src_row = cid * EROWS + base_row

        hf = ch // 2
        hfe = hf * 128
        halves_sems = ((0, sem_sa), (hf, sem_sb))

        def scale_half(h0):
            def scale_row(j, c0):
                def scale_grp(g, c1):
                    w16 = wv[j, pl.ds(g * L, L)]
                    for e in range(L):
                        wb = _lane_bcast(w16, e)
                        r = j * 128 + g * L + e
                        for hh in range(halves):
                            sl = pl.ds(hh * L, L)
                            rows[r, sl] = rows[r, sl] * wb
                    return c1
                return lax.fori_loop(0, 128 // L, scale_grp, c0)
            lax.fori_loop(h0, h0 + hf, scale_row, 0)

        def chunk(k, carry):
            r0 = base_row + k * ch
            rs = src_row + k * ch
            gd = []
            for h0, sem_s in halves_sems:
                @pl.when(k > 0)
                def _drain(h0=h0, sem_s=sem_s):
                    pltpu.make_async_copy(
                        h_hbm.at[pl.ds(0, hfe)],
                        rows.at[pl.ds(h0 * 128, hfe)], sem_s).wait()
                d1 = pltpu.async_copy(src_hbm.at[pl.ds(rs + h0, hf)],
                                      idxs.at[pl.ds(h0, hf)], sem_i)
                d2 = pltpu.async_copy(dst_hbm.at[pl.ds(r0 + h0, hf)],
                                      idxd.at[pl.ds(h0, hf)], sem_i)
                d3 = pltpu.async_copy(w_hbm.at[pl.ds(r0 + h0, hf)],
                                      wv.at[pl.ds(h0, hf)], sem_i)
                d1.wait(); d2.wait(); d3.wait()
                gd.append([
                    pltpu.async_copy(h_hbm.at[idxs.at[j]],
                                     rows.at[pl.ds(j * 128, 128)], sem_g)
                    for j in range(h0, h0 + hf)
                ])
            for (h0, sem_s), descs in zip(halves_sems, gd):
                for d in descs:
                    d.wait()
                scale_half(h0)
                for j in range(h0, h0 + hf):
                    pltpu.async_copy(rows.at[pl.ds(j * 128, 128)],
                                     acc.at[idxd.at[j]], sem_s, add=True)
            return carry

        lax.fori_loop(0, n_chunks, chunk, 0)
        for h0, sem_s in halves_sems:
            pltpu.make_async_copy(h_hbm.at[pl.ds(0, hfe)],
                                  rows.at[pl.ds(h0 * 128, hfe)], sem_s).wait()
        plsc.subcore_barrier()

        r0 = sid * ROWS_PT
        pltpu.sync_copy(acc.at[pl.ds(r0, ROWS_PT)],
                        out_hbm.at[pl.ds(cid * NPAD + r0, ROWS_PT)])

    return agg_kernel


BN = 2048
NB = NPAD // BN


def _l0_body(agg_ref, x_ref, wrl, wrh, wsl, wsh, bl, bh, out_ref):
    a = agg_ref[0] + agg_ref[1]
    x = x_ref[...]
    out_ref[0] = jnp.maximum(
        jnp.dot(a, wrl[...], preferred_element_type=jnp.float32)
        + jnp.dot(x, wsl[...], preferred_element_type=jnp.float32) + bl[...], 0.0)
    out_ref[1] = jnp.maximum(
        jnp.dot(a, wrh[...], preferred_element_type=jnp.float32)
        + jnp.dot(x, wsh[...], preferred_element_type=jnp.float32) + bh[...], 0.0)


def _conv_body(agg_ref, h_ref, wr_ll, wr_lh, wr_hl, wr_hh,
               ws_ll, ws_lh, ws_hl, ws_hh, bl, bh, out_ref):
    a0, a1 = agg_ref[0], agg_ref[1]
    h0, h1 = h_ref[0], h_ref[1]
    dot = lambda m, w: jnp.dot(m, w[...], preferred_element_type=jnp.float32)
    out_ref[0] = jnp.maximum(
        dot(a0, wr_ll) + dot(a1, wr_hl) + dot(h0, ws_ll) + dot(h1, ws_hl)
        + bl[...], 0.0)
    out_ref[1] = jnp.maximum(
        dot(a0, wr_lh) + dot(a1, wr_hh) + dot(h0, ws_lh) + dot(h1, ws_hh)
        + bh[...], 0.0)


def _sigmoid(x):
    return 1.0 / (1.0 + jnp.exp(-x))


def _head_body(h_ref, batch_ref,
               wi, wf, wg, wo, ui, uf, ug, uo, bi, bf, bg, bo,
               w0, b0, w1, b1, wout, bout,
               out_ref, sums, cnt):
    i = pl.program_id(0)

    @pl.when(i == 0)
    def _init():
        sums[...] = jnp.zeros_like(sums)
        cnt[...] = jnp.zeros_like(cnt)

    hcat = jnp.concatenate([h_ref[0], h_ref[1]], axis=1)
    brow = batch_ref[0]
    oh = (lax.broadcasted_iota(jnp.int32, (128, BN), 0) == brow
          ).astype(jnp.float32)
    sums[...] += jnp.dot(oh, hcat, preferred_element_type=jnp.float32)
    cnt[...] += jnp.dot(oh, jnp.ones((BN, 8), jnp.float32),
                        preferred_element_type=jnp.float32)

    @pl.when(i == NB - 1)
    def _head():
        pooled = sums[...] / jnp.maximum(cnt[:, 0:1], 1.0)
        riota = lax.broadcasted_iota(jnp.int32, (8, 128), 0)
        ciota = lax.broadcasted_iota(jnp.int32, (8, 128), 1)
        dot = lambda m, w: jnp.dot(m, w[...], preferred_element_type=jnp.float32)
        hh = jnp.zeros((8, 64), jnp.float32)
        cc = jnp.zeros((8, 64), jnp.float32)
        for t in range(REPS):
            sel = (riota * REPS + t == ciota).astype(jnp.float32)
            xt = jnp.dot(sel, pooled, preferred_element_type=jnp.float32)
            gi = _sigmoid(dot(xt, wi) + dot(hh, ui) + bi[...])
            gf = _sigmoid(dot(xt, wf) + dot(hh, uf) + bf[...])
            gg = jnp.tanh(dot(xt, wg) + dot(hh, ug) + bg[...])
            go = _sigmoid(dot(xt, wo) + dot(hh, uo) + bo[...])
            cc = gf * cc + gi * gg
            hh = go * jnp.tanh(cc)
            y = dot(hh, w0) + b0[...]
            y = dot(y, w1) + b1[...]
            y = dot(y, wout) + bout[...]
            out_ref[pl.ds(8 * t, 8), :] = y


def _full_spec(shape):
    return pl.BlockSpec(shape, lambda i: tuple(0 for _ in shape))


def _tc_layer0(agg0, x16, wrl, wrh, wsl, wsh, bl, bh):
    return pl.pallas_call(
        _l0_body,
        grid=(NB,),
        in_specs=[
            pl.BlockSpec((2, BN, 16), lambda i: (0, i, 0)),
            pl.BlockSpec((BN, 16), lambda i: (i, 0)),
            _full_spec((16, 32)), _full_spec((16, 32)),
            _full_spec((16, 32)), _full_spec((16, 32)),
            _full_spec((1, 32)), _full_spec((1, 32)),
        ],
        out_specs=pl.BlockSpec((2, BN, 32), lambda i: (0, i, 0)),
        out_shape=jax.ShapeDtypeStruct((2, NPAD, 32), jnp.float32),
    )(agg0, x16, wrl, wrh, wsl, wsh, bl, bh)


def _tc_conv(agg, h, wq, sq, bl, bh):
    return pl.pallas_call(
        _conv_body,
        grid=(NB,),
        in_specs=[
            pl.BlockSpec((2, BN, 32), lambda i: (0, i, 0)),
            pl.BlockSpec((2, BN, 32), lambda i: (0, i, 0)),
        ] + [_full_spec((32, 32))] * 8 + [_full_spec((1, 32))] * 2,
        out_specs=pl.BlockSpec((2, BN, 32), lambda i: (0, i, 0)),
        out_shape=jax.ShapeDtypeStruct((2, NPAD, 32), jnp.float32),
    )(agg, h, *wq, *sq, bl, bh)


def _tc_head(h, batch2d, lstm_w, mlp_w):
    return pl.pallas_call(
        _head_body,
        grid=(NB,),
        in_specs=[
            pl.BlockSpec((2, BN, 32), lambda i: (0, i, 0)),
            pl.BlockSpec((1, 1, BN), lambda i: (i, 0, 0)),
        ] + [_full_spec((64, 64))] * 8 + [_full_spec((1, 64))] * 4
          + [_full_spec((64, 64)), _full_spec((1, 64)),
             _full_spec((64, 32)), _full_spec((1, 32)),
             _full_spec((32, 128)), _full_spec((1, 128))],
        out_specs=pl.BlockSpec((80, 128), lambda i: (0, 0)),
        out_shape=jax.ShapeDtypeStruct((80, 128), jnp.float32),
        scratch_shapes=[
            pltpu.VMEM((128, 64), jnp.float32),
            pltpu.VMEM((128, 8), jnp.float32),
        ],
    )(h, batch2d, *lstm_w, *mlp_w)


def kernel(x, edge_index, edge_attr, batch,
           gW_rel_0, gb_0, gW_root_0,
           gW_rel_1, gb_1, gW_root_1,
           gW_rel_2, gb_2, gW_root_2,
           gW_rel_3, gb_3, gW_root_3,
           lstm_Wih, lstm_Whh, lstm_bih, lstm_bhh,
           mlp_W0, mlp_b0, mlp_W1, mlp_b1, out_W, out_b):
    f32 = jnp.float32
    src = edge_index[0]
    dst = edge_index[1]
    npad = EPAD - E
    pad_idx = (jnp.arange(npad, dtype=jnp.int32) * 41) % N
    srcp = jnp.concatenate([src, pad_idx])
    dstp = jnp.concatenate([dst, pad_idx])
    wp = jnp.concatenate([edge_attr, jnp.zeros((npad,), f32)])
    src2 = jnp.concatenate([srcp, srcp + NPAD]).reshape(2 * EROWS, 128)
    dst2 = dstp.reshape(EROWS, 128)
    w2 = wp.reshape(EROWS, 128)

    x16 = jnp.pad(x, ((0, NPAD - N), (0, 16 - x.shape[1])))
    z16 = jnp.zeros((ROWS_PT, 16), f32)
    z32 = jnp.zeros((ROWS_PT, 32), f32)

    agg16_fn = _make_agg(16, split_edges=True)
    agg32_fn = _make_agg(32, split_edges=False)

    agg0 = agg16_fn(x16, src2, dst2, w2, z16).reshape(2, NPAD, 16)
    wr0 = jnp.pad(gW_rel_0, ((0, 13), (0, 0)))
    ws0 = jnp.pad(gW_root_0, ((0, 13), (0, 0)))
    h = _tc_layer0(agg0, x16,
                   wr0[:, :32], wr0[:, 32:], ws0[:, :32], ws0[:, 32:],
                   gb_0[:32].reshape(1, 32), gb_0[32:].reshape(1, 32))

    for wr, ws, b in ((gW_rel_1, gW_root_1, gb_1),
                      (gW_rel_2, gW_root_2, gb_2),
                      (gW_rel_3, gW_root_3, gb_3)):
        agg = agg32_fn(h.reshape(2 * NPAD, 32), src2, dst2, w2, z32)
        wq = (wr[:32, :32], wr[:32, 32:], wr[32:, :32], wr[32:, 32:])
        sq = (ws[:32, :32], ws[:32, 32:], ws[32:, :32], ws[32:, 32:])
        h = _tc_conv(agg.reshape(2, NPAD, 32), h, wq, sq,
                     b[:32].reshape(1, 32), b[32:].reshape(1, 32))

    wih_t = lstm_Wih.T
    whh_t = lstm_Whh.T
    bsum = (lstm_bih + lstm_bhh).reshape(1, 256)
    lstm_w = (wih_t[:, 0:64], wih_t[:, 64:128], wih_t[:, 128:192],
              wih_t[:, 192:256],
              whh_t[:, 0:64], whh_t[:, 64:128], whh_t[:, 128:192],
              whh_t[:, 192:256],
              bsum[:, 0:64], bsum[:, 64:128], bsum[:, 128:192],
              bsum[:, 192:256])
    wout = jnp.zeros((32, 128), f32).at[:, 0:1].set(out_W)
    bout = jnp.zeros((1, 128), f32).at[0, 0].set(out_b[0])
    mlp_w = (mlp_W0, mlp_b0.reshape(1, 64), mlp_W1, mlp_b1.reshape(1, 32),
             wout, bout)
    batch_pad = jnp.concatenate(
        [batch, jnp.full((NPAD - N,), 100, jnp.int32)])
    out80 = _tc_head(h, batch_pad.reshape(NB, 1, BN), lstm_w, mlp_w)

    y = out80.reshape(REPS, 8, 128)[:, :NG // REPS, 0]
    return jnp.transpose(y)[:, :, None]


# speedup vs baseline: 10.7351x; 1.1668x over previous
import functools

import jax
import jax.numpy as jnp
from jax import lax
from jax.experimental import pallas as pl
from jax.experimental.pallas import tpu as pltpu
from jax.experimental.pallas import tpu_sc as plsc

N = 50000
NPAD = 51200
E = 800000
EPAD = 819200
NG = 50
REPS = 10
NC, NS, L = 2, 16, 16
ROWS_PT = NPAD // NS
EROWS = EPAD // 128

_GD = lax.GatherDimensionNumbers(
    offset_dims=(), collapsed_slice_dims=(0,), start_index_map=(0,))


def _lane_bcast(v, e):
    idx = jnp.full((L, 1), e, jnp.int32)
    return lax.gather(v, idx, _GD, (1,),
                      mode=lax.GatherScatterMode.PROMISE_IN_BOUNDS)


def _make_agg(width, split_edges):
    if split_edges:
        rows_per_tile = EROWS // (NC * NS)
        ch = 8
    else:
        rows_per_tile = EROWS // NS
        ch = 4
    n_chunks = rows_per_tile // ch
    halves = width // L

    mesh = plsc.VectorSubcoreMesh(core_axis_name="c", subcore_axis_name="s")

    @functools.partial(
        pl.kernel,
        out_type=jax.ShapeDtypeStruct((NC * NPAD, width), jnp.float32),
        mesh=mesh,
        compiler_params=pltpu.CompilerParams(use_tc_tiling_on_sc=False),
        scratch_types=[
            pltpu.VMEM_SHARED((NPAD, width), jnp.float32),
            pltpu.VMEM((ch, 128), jnp.int32),
            pltpu.VMEM((ch, 128), jnp.int32),
            pltpu.VMEM((ch, 128), jnp.float32),
            pltpu.VMEM((ch * 128, width), jnp.float32),
            pltpu.SemaphoreType.DMA,
            pltpu.SemaphoreType.DMA,
            pltpu.SemaphoreType.DMA,
            pltpu.SemaphoreType.DMA,
        ],
    )
    def agg_kernel(h_hbm, src_hbm, dst_hbm, w_hbm, z_hbm, out_hbm,
                   acc, idxs, idxd, wv, rows, sem_i, sem_g, sem_sa, sem_sb):
        cid = lax.axis_index("c")
        sid = lax.axis_index("s")

        pltpu.sync_copy(z_hbm, acc.at[pl.ds(sid * ROWS_PT, ROWS_PT)])
        plsc.subcore_barrier()

        if split_edges:
            wid = cid * NS + sid
            base_row = wid * rows_per_tile
            src_row = base_row
        else:
            base_row = sid * rows_per_tile
            src_row = cid * EROWS + base_row

        hf = ch // 2
        hfe = hf * 128
        halves_sems = ((0, sem_sa), (hf, sem_sb))

        def scale_half(h0):
            def scale_row(j, c0):
                def scale_grp(g, c1):
                    w16 = wv[j, pl.ds(g * L, L)]
                    for e in range(L):
                        wb = _lane_bcast(w16, e)
                        r = j * 128 + g * L + e
                        for hh in range(halves):
                            sl = pl.ds(hh * L, L)
                            rows[r, sl] = rows[r, sl] * wb
                    return c1
                return lax.fori_loop(0, 128 // L, scale_grp, c0)
            lax.fori_loop(h0, h0 + hf, scale_row, 0)

        def chunk(k, carry):
            r0 = base_row + k * ch
            rs = src_row + k * ch
            gd = []
            for h0, sem_s in halves_sems:
                @pl.when(k > 0)
                def _drain(h0=h0, sem_s=sem_s):
                    pltpu.make_async_copy(
                        h_hbm.at[pl.ds(0, hfe)],
                        rows.at[pl.ds(h0 * 128, hfe)], sem_s).wait()
                d1 = pltpu.async_copy(src_hbm.at[pl.ds(rs + h0, hf)],
                                      idxs.at[pl.ds(h0, hf)], sem_i)
                d2 = pltpu.async_copy(dst_hbm.at[pl.ds(r0 + h0, hf)],
                                      idxd.at[pl.ds(h0, hf)], sem_i)
                d3 = pltpu.async_copy(w_hbm.at[pl.ds(r0 + h0, hf)],
                                      wv.at[pl.ds(h0, hf)], sem_i)
                d1.wait(); d2.wait(); d3.wait()
                gd.append([
                    pltpu.async_copy(h_hbm.at[idxs.at[j]],
                                     rows.at[pl.ds(j * 128, 128)], sem_g)
                    for j in range(h0, h0 + hf)
                ])
            for (h0, sem_s), descs in zip(halves_sems, gd):
                for d in descs:
                    d.wait()
                scale_half(h0)
                for j in range(h0, h0 + hf):
                    pltpu.async_copy(rows.at[pl.ds(j * 128, 128)],
                                     acc.at[idxd.at[j]], sem_s, add=True)
            return carry

        lax.fori_loop(0, n_chunks, chunk, 0)
        for h0, sem_s in halves_sems:
            pltpu.make_async_copy(h_hbm.at[pl.ds(0, hfe)],
                                  rows.at[pl.ds(h0 * 128, hfe)], sem_s).wait()
        plsc.subcore_barrier()

        r0 = sid * ROWS_PT
        pltpu.sync_copy(acc.at[pl.ds(r0, ROWS_PT)],
                        out_hbm.at[pl.ds(cid * NPAD + r0, ROWS_PT)])

    return agg_kernel


BN = 2048
NB = NPAD // BN


def _l0_body(agg_ref, x_ref, wrl, wrh, wsl, wsh, bl, bh, out_ref):
    a = agg_ref[0] + agg_ref[1]
    x = x_ref[...]
    out_ref[0] = jnp.maximum(
        jnp.dot(a, wrl[...], preferred_element_type=jnp.float32)
        + jnp.dot(x, wsl[...], preferred_element_type=jnp.float32) + bl[...], 0.0)
    out_ref[1] = jnp.maximum(
        jnp.dot(a, wrh[...], preferred_element_type=jnp.float32)
        + jnp.dot(x, wsh[...], preferred_element_type=jnp.float32) + bh[...], 0.0)


def _conv_body(agg_ref, h_ref, wr_ll, wr_lh, wr_hl, wr_hh,
               ws_ll, ws_lh, ws_hl, ws_hh, bl, bh, out_ref):
    a0, a1 = agg_ref[0], agg_ref[1]
    h0, h1 = h_ref[0], h_ref[1]
    dot = lambda m, w: jnp.dot(m, w[...], preferred_element_type=jnp.float32)
    out_ref[0] = jnp.maximum(
        dot(a0, wr_ll) + dot(a1, wr_hl) + dot(h0, ws_ll) + dot(h1, ws_hl)
        + bl[...], 0.0)
    out_ref[1] = jnp.maximum(
        dot(a0, wr_lh) + dot(a1, wr_hh) + dot(h0, ws_lh) + dot(h1, ws_hh)
        + bh[...], 0.0)


def _sigmoid(x):
    return 1.0 / (1.0 + jnp.exp(-x))


def _head_body(h_ref, batch_ref,
               wi, wf, wg, wo, ui, uf, ug, uo, bi, bf, bg, bo,
               w0, b0, w1, b1, wout, bout,
               out_ref, sums, cnt):
    i = pl.program_id(0)

    @pl.when(i == 0)
    def _init():
        sums[...] = jnp.zeros_like(sums)
        cnt[...] = jnp.zeros_like(cnt)

    hcat = jnp.concatenate([h_ref[0], h_ref[1]], axis=1)
    brow = batch_ref[0]
    oh = (lax.broadcasted_iota(jnp.int32, (128, BN), 0) == brow
          ).astype(jnp.float32)
    sums[...] += jnp.dot(oh, hcat, preferred_element_type=jnp.float32)
    cnt[...] += jnp.dot(oh, jnp.ones((BN, 8), jnp.float32),
                        preferred_element_type=jnp.float32)

    @pl.when(i == NB - 1)
    def _head():
        pooled = sums[...] / jnp.maximum(cnt[:, 0:1], 1.0)
        riota = lax.broadcasted_iota(jnp.int32, (8, 128), 0)
        ciota = lax.broadcasted_iota(jnp.int32, (8, 128), 1)
        dot = lambda m, w: jnp.dot(m, w[...], preferred_element_type=jnp.float32)
        hh = jnp.zeros((8, 64), jnp.float32)
        cc = jnp.zeros((8, 64), jnp.float32)
        for t in range(REPS):
            sel = (riota * REPS + t == ciota).astype(jnp.float32)
            xt = jnp.dot(sel, pooled, preferred_element_type=jnp.float32)
            gi = _sigmoid(dot(xt, wi) + dot(hh, ui) + bi[...])
            gf = _sigmoid(dot(xt, wf) + dot(hh, uf) + bf[...])
            gg = jnp.tanh(dot(xt, wg) + dot(hh, ug) + bg[...])
            go = _sigmoid(dot(xt, wo) + dot(hh, uo) + bo[...])
            cc = gf * cc + gi * gg
            hh = go * jnp.tanh(cc)
            y = dot(hh, w0) + b0[...]
            y = dot(y, w1) + b1[...]
            y = dot(y, wout) + bout[...]
            out_ref[pl.ds(8 * t, 8), :] = y


def _full_spec(shape):
    return pl.BlockSpec(shape, lambda i: tuple(0 for _ in shape))


def _tc_layer0(agg0, x16, wrl, wrh, wsl, wsh, bl, bh):
    return pl.pallas_call(
        _l0_body,
        grid=(NB,),
        in_specs=[
            pl.BlockSpec((2, BN, 16), lambda i: (0, i, 0)),
            pl.BlockSpec((BN, 16), lambda i: (i, 0)),
            _full_spec((16, 32)), _full_spec((16, 32)),
            _full_spec((16, 32)), _full_spec((16, 32)),
            _full_spec((1, 32)), _full_spec((1, 32)),
        ],
        out_specs=pl.BlockSpec((2, BN, 32), lambda i: (0, i, 0)),
        out_shape=jax.ShapeDtypeStruct((2, NPAD, 32), jnp.float32),
    )(agg0, x16, wrl, wrh, wsl, wsh, bl, bh)


def _tc_conv(agg, h, wq, sq, bl, bh):
    return pl.pallas_call(
        _conv_body,
        grid=(NB,),
        in_specs=[
            pl.BlockSpec((2, BN // 4, 128), lambda i: (0, i, 0)),
            pl.BlockSpec((2, BN // 4, 128), lambda i: (0, i, 0)),
        ] + [_full_spec((128, 128))] * 8 + [_full_spec((1, 128))] * 2,
        out_specs=pl.BlockSpec((2, BN // 4, 128), lambda i: (0, i, 0)),
        out_shape=jax.ShapeDtypeStruct((2, NPAD // 4, 128), jnp.float32),
    )(agg, h, *wq, *sq, bl, bh)


def _tc_head(h, batch2d, lstm_w, mlp_w):
    return pl.pallas_call(
        _head_body,
        grid=(NB,),
        in_specs=[
            pl.BlockSpec((2, BN, 32), lambda i: (0, i, 0)),
            pl.BlockSpec((1, 1, BN), lambda i: (i, 0, 0)),
        ] + [_full_spec((64, 64))] * 8 + [_full_spec((1, 64))] * 4
          + [_full_spec((64, 64)), _full_spec((1, 64)),
             _full_spec((64, 32)), _full_spec((1, 32)),
             _full_spec((32, 128)), _full_spec((1, 128))],
        out_specs=pl.BlockSpec((80, 128), lambda i: (0, 0)),
        out_shape=jax.ShapeDtypeStruct((80, 128), jnp.float32),
        scratch_shapes=[
            pltpu.VMEM((128, 64), jnp.float32),
            pltpu.VMEM((128, 8), jnp.float32),
        ],
    )(h, batch2d, *lstm_w, *mlp_w)


def kernel(x, edge_index, edge_attr, batch,
           gW_rel_0, gb_0, gW_root_0,
           gW_rel_1, gb_1, gW_root_1,
           gW_rel_2, gb_2, gW_root_2,
           gW_rel_3, gb_3, gW_root_3,
           lstm_Wih, lstm_Whh, lstm_bih, lstm_bhh,
           mlp_W0, mlp_b0, mlp_W1, mlp_b1, out_W, out_b):
    f32 = jnp.float32
    src = edge_index[0]
    dst = edge_index[1]
    npad = EPAD - E
    pad_idx = (jnp.arange(npad, dtype=jnp.int32) * 41) % N
    srcp = jnp.concatenate([src, pad_idx])
    dstp = jnp.concatenate([dst, pad_idx])
    wp = jnp.concatenate([edge_attr, jnp.zeros((npad,), f32)])
    src2 = jnp.concatenate([srcp, srcp + NPAD]).reshape(2 * EROWS, 128)
    dst2 = dstp.reshape(EROWS, 128)
    w2 = wp.reshape(EROWS, 128)

    x16 = jnp.pad(x, ((0, NPAD - N), (0, 16 - x.shape[1])))
    z16 = jnp.zeros((ROWS_PT, 16), f32)
    z32 = jnp.zeros((ROWS_PT, 32), f32)

    agg16_fn = _make_agg(16, split_edges=True)
    agg32_fn = _make_agg(32, split_edges=False)

    agg0 = agg16_fn(x16, src2, dst2, w2, z16).reshape(2, NPAD, 16)
    wr0 = jnp.pad(gW_rel_0, ((0, 13), (0, 0)))
    ws0 = jnp.pad(gW_root_0, ((0, 13), (0, 0)))
    h = _tc_layer0(agg0, x16,
                   wr0[:, :32], wr0[:, 32:], ws0[:, :32], ws0[:, 32:],
                   gb_0[:32].reshape(1, 32), gb_0[32:].reshape(1, 32))
    h = h.reshape(2, NPAD // 4, 128)

    eye4 = jnp.eye(4, dtype=f32)
    for wr, ws, b in ((gW_rel_1, gW_root_1, gb_1),
                      (gW_rel_2, gW_root_2, gb_2),
                      (gW_rel_3, gW_root_3, gb_3)):
        agg = agg32_fn(h.reshape(2 * NPAD, 32), src2, dst2, w2, z32)
        wq = tuple(jnp.kron(eye4, q) for q in
                   (wr[:32, :32], wr[:32, 32:], wr[32:, :32], wr[32:, 32:]))
        sq = tuple(jnp.kron(eye4, q) for q in
                   (ws[:32, :32], ws[:32, 32:], ws[32:, :32], ws[32:, 32:]))
        h = _tc_conv(agg.reshape(2, NPAD // 4, 128), h, wq, sq,
                     jnp.tile(b[:32], 4).reshape(1, 128),
                     jnp.tile(b[32:], 4).reshape(1, 128))

    wih_t = lstm_Wih.T
    whh_t = lstm_Whh.T
    bsum = (lstm_bih + lstm_bhh).reshape(1, 256)
    lstm_w = (wih_t[:, 0:64], wih_t[:, 64:128], wih_t[:, 128:192],
              wih_t[:, 192:256],
              whh_t[:, 0:64], whh_t[:, 64:128], whh_t[:, 128:192],
              whh_t[:, 192:256],
              bsum[:, 0:64], bsum[:, 64:128], bsum[:, 128:192],
              bsum[:, 192:256])
    wout = jnp.zeros((32, 128), f32).at[:, 0:1].set(out_W)
    bout = jnp.zeros((1, 128), f32).at[0, 0].set(out_b[0])
    mlp_w = (mlp_W0, mlp_b0.reshape(1, 64), mlp_W1, mlp_b1.reshape(1, 32),
             wout, bout)
    batch_pad = jnp.concatenate(
        [batch, jnp.full((NPAD - N,), 100, jnp.int32)])
    out80 = _tc_head(h.reshape(2, NPAD, 32), batch_pad.reshape(NB, 1, BN),
                     lstm_w, mlp_w)

    y = out80.reshape(REPS, 8, 128)[:, :NG // REPS, 0]
    return jnp.transpose(y)[:, :, None]


# speedup vs baseline: 11.6421x; 1.0845x over previous
import functools

import jax
import jax.numpy as jnp
from jax import lax
from jax.experimental import pallas as pl
from jax.experimental.pallas import tpu as pltpu
from jax.experimental.pallas import tpu_sc as plsc

N = 50000
NPAD = 51200
E = 800000
EPAD = 819200
NG = 50
REPS = 10
NC, NS, L = 2, 16, 16
ROWS_PT = NPAD // NS
EROWS = EPAD // 128

_GD = lax.GatherDimensionNumbers(
    offset_dims=(), collapsed_slice_dims=(0,), start_index_map=(0,))


def _lane_bcast(v, e):
    idx = jnp.full((L, 1), e, jnp.int32)
    return lax.gather(v, idx, _GD, (1,),
                      mode=lax.GatherScatterMode.PROMISE_IN_BOUNDS)


def _make_agg(width, split_edges):
    if split_edges:
        rows_per_tile = EROWS // (NC * NS)
        ch = 8
    else:
        rows_per_tile = EROWS // NS
        ch = 4
    n_chunks = rows_per_tile // ch
    halves = width // L

    mesh = plsc.VectorSubcoreMesh(core_axis_name="c", subcore_axis_name="s")

    @functools.partial(
        pl.kernel,
        out_type=jax.ShapeDtypeStruct((NC * NPAD, width), jnp.float32),
        mesh=mesh,
        compiler_params=pltpu.CompilerParams(use_tc_tiling_on_sc=False),
        scratch_types=[
            pltpu.VMEM_SHARED((NPAD, width), jnp.float32),
            pltpu.VMEM((ch, 128), jnp.int32),
            pltpu.VMEM((ch, 128), jnp.int32),
            pltpu.VMEM((ch, 128), jnp.float32),
            pltpu.VMEM((ch * 128, width), jnp.float32),
            pltpu.SemaphoreType.DMA,
            pltpu.SemaphoreType.DMA,
            pltpu.SemaphoreType.DMA,
            pltpu.SemaphoreType.DMA,
        ],
    )
    def agg_kernel(h_hbm, src_hbm, dst_hbm, w_hbm, z_hbm, out_hbm,
                   acc, idxs, idxd, wv, rows, sem_i, sem_g, sem_sa, sem_sb):
        cid = lax.axis_index("c")
        sid = lax.axis_index("s")

        pltpu.sync_copy(z_hbm, acc.at[pl.ds(sid * ROWS_PT, ROWS_PT)])
        plsc.subcore_barrier()

        if split_edges:
            wid = cid * NS + sid
            base_row = wid * rows_per_tile
            src_row = base_row
        else:
            base_row = sid * rows_per_tile
            src_row = cid * EROWS + base_row

        hf = ch // 2
        hfe = hf * 128
        halves_sems = ((0, sem_sa), (hf, sem_sb))

        def scale_half(h0):
            def scale_row(j, c0):
                def scale_grp(g, c1):
                    w16 = wv[j, pl.ds(g * L, L)]
                    for e in range(L):
                        wb = _lane_bcast(w16, e)
                        r = j * 128 + g * L + e
                        for hh in range(halves):
                            sl = pl.ds(hh * L, L)
                            rows[r, sl] = rows[r, sl] * wb
                    return c1
                return lax.fori_loop(0, 128 // L, scale_grp, c0)
            lax.fori_loop(h0, h0 + hf, scale_row, 0)

        def chunk(k, carry):
            r0 = base_row + k * ch
            rs = src_row + k * ch
            gd = []
            for h0, sem_s in halves_sems:
                @pl.when(k > 0)
                def _drain(h0=h0, sem_s=sem_s):
                    pltpu.make_async_copy(
                        h_hbm.at[pl.ds(0, hfe)],
                        rows.at[pl.ds(h0 * 128, hfe)], sem_s).wait()
                d1 = pltpu.async_copy(src_hbm.at[pl.ds(rs + h0, hf)],
                                      idxs.at[pl.ds(h0, hf)], sem_i)
                d2 = pltpu.async_copy(dst_hbm.at[pl.ds(r0 + h0, hf)],
                                      idxd.at[pl.ds(h0, hf)], sem_i)
                d3 = pltpu.async_copy(w_hbm.at[pl.ds(r0 + h0, hf)],
                                      wv.at[pl.ds(h0, hf)], sem_i)
                d1.wait(); d2.wait(); d3.wait()
                gd.append([
                    pltpu.async_copy(h_hbm.at[idxs.at[j]],
                                     rows.at[pl.ds(j * 128, 128)], sem_g)
                    for j in range(h0, h0 + hf)
                ])
            for (h0, sem_s), descs in zip(halves_sems, gd):
                for d in descs:
                    d.wait()
                scale_half(h0)
                for j in range(h0, h0 + hf):
                    pltpu.async_copy(rows.at[pl.ds(j * 128, 128)],
                                     acc.at[idxd.at[j]], sem_s, add=True)
            return carry

        lax.fori_loop(0, n_chunks, chunk, 0)
        for h0, sem_s in halves_sems:
            pltpu.make_async_copy(h_hbm.at[pl.ds(0, hfe)],
                                  rows.at[pl.ds(h0 * 128, hfe)], sem_s).wait()
        plsc.subcore_barrier()

        r0 = sid * ROWS_PT
        pltpu.sync_copy(acc.at[pl.ds(r0, ROWS_PT)],
                        out_hbm.at[pl.ds(cid * NPAD + r0, ROWS_PT)])

    return agg_kernel


BN = 2048
NB = NPAD // BN


def _l0_body(agg_ref, x_ref, wrl, wrh, wsl, wsh, bl, bh, out_ref):
    a = agg_ref[0] + agg_ref[1]
    x = x_ref[...]
    out_ref[0] = jnp.maximum(
        jnp.dot(a, wrl[...], preferred_element_type=jnp.float32, precision=lax.Precision.HIGHEST)
        + jnp.dot(x, wsl[...], preferred_element_type=jnp.float32, precision=lax.Precision.HIGHEST) + bl[...], 0.0)
    out_ref[1] = jnp.maximum(
        jnp.dot(a, wrh[...], preferred_element_type=jnp.float32, precision=lax.Precision.HIGHEST)
        + jnp.dot(x, wsh[...], preferred_element_type=jnp.float32, precision=lax.Precision.HIGHEST) + bh[...], 0.0)


def _conv_body(agg_ref, h_ref, wr_ll, wr_lh, wr_hl, wr_hh,
               ws_ll, ws_lh, ws_hl, ws_hh, bl, bh, out_ref):
    a0, a1 = agg_ref[0], agg_ref[1]
    h0, h1 = h_ref[0], h_ref[1]
    dot = lambda m, w: jnp.dot(m, w[...], preferred_element_type=jnp.float32, precision=lax.Precision.HIGHEST)
    out_ref[0] = jnp.maximum(
        dot(a0, wr_ll) + dot(a1, wr_hl) + dot(h0, ws_ll) + dot(h1, ws_hl)
        + bl[...], 0.0)
    out_ref[1] = jnp.maximum(
        dot(a0, wr_lh) + dot(a1, wr_hh) + dot(h0, ws_lh) + dot(h1, ws_hh)
        + bh[...], 0.0)


def _sigmoid(x):
    return 1.0 / (1.0 + jnp.exp(-x))


def _head_body(h_ref, batch_ref,
               wi, wf, wg, wo, ui, uf, ug, uo, bi, bf, bg, bo,
               w0, b0, w1, b1, wout, bout,
               out_ref, sums, cnt):
    i = pl.program_id(0)

    @pl.when(i == 0)
    def _init():
        sums[...] = jnp.zeros_like(sums)
        cnt[...] = jnp.zeros_like(cnt)

    hcat = jnp.concatenate([h_ref[0], h_ref[1]], axis=1)
    brow = batch_ref[0]
    oh = (lax.broadcasted_iota(jnp.int32, (128, BN), 0) == brow
          ).astype(jnp.float32)
    sums[...] += jnp.dot(oh, hcat, preferred_element_type=jnp.float32, precision=lax.Precision.HIGHEST)
    cnt[...] += jnp.dot(oh, jnp.ones((BN, 8), jnp.float32),
                        preferred_element_type=jnp.float32, precision=lax.Precision.HIGHEST)

    @pl.when(i == NB - 1)
    def _head():
        pooled = sums[...] / jnp.maximum(cnt[:, 0:1], 1.0)
        riota = lax.broadcasted_iota(jnp.int32, (8, 128), 0)
        ciota = lax.broadcasted_iota(jnp.int32, (8, 128), 1)
        dot = lambda m, w: jnp.dot(m, w[...], preferred_element_type=jnp.float32, precision=lax.Precision.HIGHEST)
        hh = jnp.zeros((8, 64), jnp.float32)
        cc = jnp.zeros((8, 64), jnp.float32)
        for t in range(REPS):
            sel = (riota * REPS + t == ciota).astype(jnp.float32)
            xt = jnp.dot(sel, pooled, preferred_element_type=jnp.float32, precision=lax.Precision.HIGHEST)
            gi = _sigmoid(dot(xt, wi) + dot(hh, ui) + bi[...])
            gf = _sigmoid(dot(xt, wf) + dot(hh, uf) + bf[...])
            gg = jnp.tanh(dot(xt, wg) + dot(hh, ug) + bg[...])
            go = _sigmoid(dot(xt, wo) + dot(hh, uo) + bo[...])
            cc = gf * cc + gi * gg
            hh = go * jnp.tanh(cc)
            y = dot(hh, w0) + b0[...]
            y = dot(y, w1) + b1[...]
            y = dot(y, wout) + bout[...]
            out_ref[pl.ds(8 * t, 8), :] = y


def _full_spec(shape):
    return pl.BlockSpec(shape, lambda i: tuple(0 for _ in shape))


def _tc_layer0(agg0, x16, wrl, wrh, wsl, wsh, bl, bh):
    return pl.pallas_call(
        _l0_body,
        grid=(NB,),
        in_specs=[
            pl.BlockSpec((2, BN, 16), lambda i: (0, i, 0)),
            pl.BlockSpec((BN, 16), lambda i: (i, 0)),
            _full_spec((16, 32)), _full_spec((16, 32)),
            _full_spec((16, 32)), _full_spec((16, 32)),
            _full_spec((1, 32)), _full_spec((1, 32)),
        ],
        out_specs=pl.BlockSpec((2, BN, 32), lambda i: (0, i, 0)),
        out_shape=jax.ShapeDtypeStruct((2, NPAD, 32), jnp.float32),
    )(agg0, x16, wrl, wrh, wsl, wsh, bl, bh)


def _tc_conv(agg, h, wq, sq, bl, bh):
    return pl.pallas_call(
        _conv_body,
        grid=(NB,),
        in_specs=[
            pl.BlockSpec((2, BN // 4, 128), lambda i: (0, i, 0)),
            pl.BlockSpec((2, BN // 4, 128), lambda i: (0, i, 0)),
        ] + [_full_spec((128, 128))] * 8 + [_full_spec((1, 128))] * 2,
        out_specs=pl.BlockSpec((2, BN // 4, 128), lambda i: (0, i, 0)),
        out_shape=jax.ShapeDtypeStruct((2, NPAD // 4, 128), jnp.float32),
    )(agg, h, *wq, *sq, bl, bh)


def _tc_head(h, batch2d, lstm_w, mlp_w):
    return pl.pallas_call(
        _head_body,
        grid=(NB,),
        in_specs=[
            pl.BlockSpec((2, BN, 32), lambda i: (0, i, 0)),
            pl.BlockSpec((1, 1, BN), lambda i: (i, 0, 0)),
        ] + [_full_spec((64, 64))] * 8 + [_full_spec((1, 64))] * 4
          + [_full_spec((64, 64)), _full_spec((1, 64)),
             _full_spec((64, 32)), _full_spec((1, 32)),
             _full_spec((32, 128)), _full_spec((1, 128))],
        out_specs=pl.BlockSpec((80, 128), lambda i: (0, 0)),
        out_shape=jax.ShapeDtypeStruct((80, 128), jnp.float32),
        scratch_shapes=[
            pltpu.VMEM((128, 64), jnp.float32),
            pltpu.VMEM((128, 8), jnp.float32),
        ],
    )(h, batch2d, *lstm_w, *mlp_w)


def kernel(x, edge_index, edge_attr, batch,
           gW_rel_0, gb_0, gW_root_0,
           gW_rel_1, gb_1, gW_root_1,
           gW_rel_2, gb_2, gW_root_2,
           gW_rel_3, gb_3, gW_root_3,
           lstm_Wih, lstm_Whh, lstm_bih, lstm_bhh,
           mlp_W0, mlp_b0, mlp_W1, mlp_b1, out_W, out_b):
    f32 = jnp.float32
    src = edge_index[0]
    dst = edge_index[1]
    npad = EPAD - E
    pad_idx = (jnp.arange(npad, dtype=jnp.int32) * 41) % N
    srcp = jnp.concatenate([src, pad_idx])
    dstp = jnp.concatenate([dst, pad_idx])
    wp = jnp.concatenate([edge_attr, jnp.zeros((npad,), f32)])
    src2 = jnp.concatenate([srcp, srcp + NPAD]).reshape(2 * EROWS, 128)
    dst2 = dstp.reshape(EROWS, 128)
    w2 = wp.reshape(EROWS, 128)

    x16 = jnp.pad(x, ((0, NPAD - N), (0, 16 - x.shape[1])))
    z16 = jnp.zeros((ROWS_PT, 16), f32)
    z32 = jnp.zeros((ROWS_PT, 32), f32)

    agg16_fn = _make_agg(16, split_edges=True)
    agg32_fn = _make_agg(32, split_edges=False)

    agg0 = agg16_fn(x16, src2, dst2, w2, z16).reshape(2, NPAD, 16)
    wr0 = jnp.pad(gW_rel_0, ((0, 13), (0, 0)))
    ws0 = jnp.pad(gW_root_0, ((0, 13), (0, 0)))
    h = _tc_layer0(agg0, x16,
                   wr0[:, :32], wr0[:, 32:], ws0[:, :32], ws0[:, 32:],
                   gb_0[:32].reshape(1, 32), gb_0[32:].reshape(1, 32))
    h = h.reshape(2, NPAD // 4, 128)

    eye4 = jnp.eye(4, dtype=f32)
    for wr, ws, b in ((gW_rel_1, gW_root_1, gb_1),
                      (gW_rel_2, gW_root_2, gb_2),
                      (gW_rel_3, gW_root_3, gb_3)):
        agg = agg32_fn(h.reshape(2 * NPAD, 32), src2, dst2, w2, z32)
        wq = tuple(jnp.kron(eye4, q) for q in
                   (wr[:32, :32], wr[:32, 32:], wr[32:, :32], wr[32:, 32:]))
        sq = tuple(jnp.kron(eye4, q) for q in
                   (ws[:32, :32], ws[:32, 32:], ws[32:, :32], ws[32:, 32:]))
        h = _tc_conv(agg.reshape(2, NPAD // 4, 128), h, wq, sq,
                     jnp.tile(b[:32], 4).reshape(1, 128),
                     jnp.tile(b[32:], 4).reshape(1, 128))

    wih_t = lstm_Wih.T
    whh_t = lstm_Whh.T
    bsum = (lstm_bih + lstm_bhh).reshape(1, 256)
    lstm_w = (wih_t[:, 0:64], wih_t[:, 64:128], wih_t[:, 128:192],
              wih_t[:, 192:256],
              whh_t[:, 0:64], whh_t[:, 64:128], whh_t[:, 128:192],
              whh_t[:, 192:256],
              bsum[:, 0:64], bsum[:, 64:128], bsum[:, 128:192],
              bsum[:, 192:256])
    wout = jnp.zeros((32, 128), f32).at[:, 0:1].set(out_W)
    bout = jnp.zeros((1, 128), f32).at[0, 0].set(out_b[0])
    mlp_w = (mlp_W0, mlp_b0.reshape(1, 64), mlp_W1, mlp_b1.reshape(1, 32),
             wout, bout)
    batch_pad = jnp.concatenate(
        [batch, jnp.full((NPAD - N,), 100, jnp.int32)])
    out80 = _tc_head(h.reshape(2, NPAD, 32), batch_pad.reshape(NB, 1, BN),
                     lstm_w, mlp_w)

    y = out80.reshape(REPS, 8, 128)[:, :NG // REPS, 0]
    return jnp.transpose(y)[:, :, None]
